# baseline (speedup 1.0000x reference)
import functools

import jax
import jax.numpy as jnp
from jax import lax
from jax.experimental import pallas as pl
from jax.experimental.pallas import tpu as pltpu
from jax.experimental.pallas import tpu_sc as plsc

_H = 128
_NA = 90000
_NPT = 30000
_NW = 32
_CH = 128
_CPW = 8
_NP = _NW * _CPW * _CH
_B = 5000
_BPT = _NPT // _B


def _sc_gather3(lane_x, poly_x, agent_x, l2a_idx, g2a_idx, oth_idx, dep):
    info = plsc.get_sparse_core_info()
    nc = info.num_cores
    mesh = plsc.VectorSubcoreMesh(core_axis_name="c", subcore_axis_name="s")

    scratch = [
        pltpu.VMEM((_CPW, _CH), jnp.int32),
        pltpu.VMEM((_CH, _H), jnp.float32),
        pltpu.VMEM((_CH, _H), jnp.float32),
        pltpu.VMEM((_CH, _H), jnp.float32),
        pltpu.VMEM((_CH, _H), jnp.float32),
        pltpu.SemaphoreType.DMA,
        pltpu.SemaphoreType.DMA,
    ]

    def body(lane_hbm, poly_hbm, agent_hbm, li_hbm, gi_hbm, oi_hbm,
             lo_hbm, go_hbm, oo_hbm, idx_v, b0, b1, b2, b3, gsem, wsem):
        wid = lax.axis_index("s") * nc + lax.axis_index("c")
        base = wid * (_CPW * _CH)
        bufs = (b0, b1, b2, b3)
        for tab, ih, oh in ((lane_hbm, li_hbm, lo_hbm),
                            (poly_hbm, gi_hbm, go_hbm),
                            (agent_hbm, oi_hbm, oo_hbm)):
            pltpu.sync_copy(ih.at[wid], idx_v)

            def quad(i, carry, tab=tab, oh=oh):
                j = 4 * i
                gs = [pltpu.async_copy(tab.at[idx_v.at[j + b]], bufs[b], gsem)
                      for b in range(4)]
                ws = []
                for b in range(4):
                    gs[b].wait()
                    ws.append(pltpu.async_copy(
                        bufs[b], oh.at[pl.ds(base + (j + b) * _CH, _CH)],
                        wsem))
                for w in ws:
                    w.wait()
                return carry

            lax.fori_loop(0, _CPW // 4, quad, 0)

    out_type = [jax.ShapeDtypeStruct((_NP, _H), jnp.float32)] * 3
    if dep is None:
        k = functools.partial(pl.kernel, mesh=mesh, out_type=out_type,
                              scratch_types=scratch)(body)
        return k(lane_x, poly_x, agent_x, l2a_idx, g2a_idx, oth_idx)

    def body_dep(dep_hbm, *refs):
        body(*refs)

    k = functools.partial(pl.kernel, mesh=mesh, out_type=out_type,
                          scratch_types=scratch)(body_dep)
    return k(dep, lane_x, poly_x, agent_x, l2a_idx, g2a_idx, oth_idx)


def _ln(x, g, b):
    m = jnp.mean(x, axis=-1, keepdims=True)
    v = jnp.mean((x - m) * (x - m), axis=-1, keepdims=True)
    return (x - m) * lax.rsqrt(v + 1e-5) * g + b


def _dot(a, b):
    return lax.dot_general(a, b, (((1,), (0,)), ((), ())),
                           preferred_element_type=jnp.float32)


def _dense_body(cur_ref, lane_ref, poly_ref, oth_ref,
                wself_ref, bself_ref,
                l2a_g_ref, l2a_b_ref, l2a_w1_ref, l2a_b1_ref, l2a_w2_ref,
                l2a_b2_ref,
                g2a_g_ref, g2a_b_ref, g2a_w1_ref, g2a_b1_ref, g2a_w2_ref,
                g2a_b2_ref,
                oth_g_ref, oth_b_ref, oth_w1_ref, oth_b1_ref, oth_w2_ref,
                oth_b2_ref,
                ofc_w_ref, ofc_b_ref,
                ffn_g_ref, ffn_b_ref, ffn_w1_ref, ffn_b1_ref, ffn_w3_ref,
                ffn_b3_ref, ffn_w2_ref, ffn_b2_ref,
                out_ref):
    cur = cur_ref[...]
    lane = lane_ref[...]
    poly = poly_ref[...]
    oth = oth_ref[...]

    self_out = jnp.maximum(_dot(cur, wself_ref[0]) + bself_ref[0], 0.0)

    h = _ln(jnp.concatenate([lane, cur, lane], axis=-1),
            l2a_g_ref[...], l2a_b_ref[...])
    l2a = _dot(jnp.maximum(_dot(h, l2a_w1_ref[...]) + l2a_b1_ref[...], 0.0),
               l2a_w2_ref[...]) + l2a_b2_ref[...]

    h = _ln(jnp.concatenate([poly, cur], axis=-1),
            g2a_g_ref[...], g2a_b_ref[...])
    g2a = _dot(jnp.maximum(_dot(h, g2a_w1_ref[...]) + g2a_b1_ref[...], 0.0),
               g2a_w2_ref[...]) + g2a_b2_ref[...]

    h = _ln(jnp.concatenate([oth, cur, oth], axis=-1),
            oth_g_ref[0], oth_b_ref[0])
    oth_out = _dot(jnp.maximum(_dot(h, oth_w1_ref[0]) + oth_b1_ref[0], 0.0),
                   oth_w2_ref[0]) + oth_b2_ref[0]

    combined = jnp.concatenate([self_out, l2a, g2a, oth_out], axis=-1)
    x = _dot(combined, ofc_w_ref[0]) + ofc_b_ref[0]

    h = _ln(x, ffn_g_ref[0], ffn_b_ref[0])
    a = _dot(h, ffn_w1_ref[0]) + ffn_b1_ref[0]
    g = _dot(h, ffn_w3_ref[0]) + ffn_b3_ref[0]
    out_ref[...] = _dot(a * lax.logistic(a) * g, ffn_w2_ref[0]) \
        + ffn_b2_ref[0] + x


def _t3(v):
    return v.reshape(3, 1, -1)


def _dense_call(b0, cur, lane_g, poly_g, oth_g,
                W_self, b_self,
                l2a_g, l2a_b, l2a_w1, l2a_b1, l2a_w2, l2a_b2,
                g2a_g, g2a_b, g2a_w1, g2a_b1, g2a_w2, g2a_b2,
                oth_lg, oth_lb, oth_w1, oth_b1, oth_w2, oth_b2,
                ofc_w, ofc_b,
                ffn_g, ffn_b, ffn_w1, ffn_b1, ffn_w3, ffn_b3, ffn_w2, ffn_b2):
    rows = lambda i: (i, 0)
    rows_o = lambda i, b0=b0: (i + b0, 0)
    typ3 = lambda i, b0=b0: ((i + b0) // _BPT, 0, 0)
    full2 = lambda i: (0, 0)
    bs = pl.BlockSpec
    tvec = lambda n: bs((1, 1, n), typ3)

    in_specs = [
        bs((_B, _H), rows_o),
        bs((_B, _H), rows),
        bs((_B, _H), rows),
        bs((_B, _H), rows),
        bs((1, _H, _H), typ3),
        tvec(_H),
        bs((1, 3 * _H), full2),
        bs((1, 3 * _H), full2),
        bs((3 * _H, 4 * _H), full2),
        bs((1, 4 * _H), full2),
        bs((4 * _H, _H), full2),
        bs((1, _H), full2),
        bs((1, 2 * _H), full2),
        bs((1, 2 * _H), full2),
        bs((2 * _H, 4 * _H), full2),
        bs((1, 4 * _H), full2),
        bs((4 * _H, _H), full2),
        bs((1, _H), full2),
        tvec(3 * _H),
        tvec(3 * _H),
        bs((1, 3 * _H, 4 * _H), typ3),
        tvec(4 * _H),
        bs((1, 4 * _H, _H), typ3),
        tvec(_H),
        bs((1, 4 * _H, _H), typ3),
        tvec(_H),
        tvec(_H),
        tvec(_H),
        bs((1, _H, 4 * _H), typ3),
        tvec(4 * _H),
        bs((1, _H, 4 * _H), typ3),
        tvec(4 * _H),
        bs((1, 4 * _H, _H), typ3),
        tvec(_H),
    ]
    return pl.pallas_call(
        _dense_body,
        grid=(_BPT,),
        in_specs=in_specs,
        out_specs=bs((_B, _H), rows),
        out_shape=jax.ShapeDtypeStruct((_NPT, _H), jnp.float32),
        compiler_params=pltpu.CompilerParams(
            dimension_semantics=("arbitrary",)),
    )(cur, lane_g, poly_g, oth_g,
      W_self, _t3(b_self),
      l2a_g.reshape(1, -1), l2a_b.reshape(1, -1), l2a_w1,
      l2a_b1.reshape(1, -1), l2a_w2, l2a_b2.reshape(1, -1),
      g2a_g.reshape(1, -1), g2a_b.reshape(1, -1), g2a_w1,
      g2a_b1.reshape(1, -1), g2a_w2, g2a_b2.reshape(1, -1),
      _t3(oth_lg), _t3(oth_lb), oth_w1, _t3(oth_b1), oth_w2, _t3(oth_b2),
      ofc_w, _t3(ofc_b), _t3(ffn_g), _t3(ffn_b),
      ffn_w1, _t3(ffn_b1), ffn_w3, _t3(ffn_b3), ffn_w2, _t3(ffn_b2))


def _pad_idx(idx):
    idx = idx.astype(jnp.int32)
    pad = jnp.zeros((_NP - _NPT,), jnp.int32)
    return jnp.concatenate([idx, pad]).reshape(_NW, _CPW, _CH)


def kernel(agent_x, lane_x, polygon_x, W_self, b_self, l2a_ln_g, l2a_ln_b,
           l2a_w1, l2a_b1, l2a_w2, l2a_b2, g2a_ln_g, g2a_ln_b, g2a_w1,
           g2a_b1, g2a_w2, g2a_b2, oth_ln_g, oth_ln_b, oth_w1, oth_b1,
           oth_w2, oth_b2, out_fc_W, out_fc_b, ffn_ln_g, ffn_ln_b, ffn_w1,
           ffn_b1, ffn_w3, ffn_b3, ffn_w2, ffn_b2, l2a_src, g2a_src,
           other_src):
    gs = []
    dep = None
    for t in range(3):
        lo, hi = t * _NPT, (t + 1) * _NPT
        g = _sc_gather3(lane_x, polygon_x, agent_x,
                        _pad_idx(l2a_src[lo:hi]),
                        _pad_idx(g2a_src[lo:hi]),
                        _pad_idx(other_src[lo:hi]), dep)
        dep = g[0]
        gs.append(g)
    weights = (W_self, b_self,
               l2a_ln_g, l2a_ln_b, l2a_w1, l2a_b1, l2a_w2, l2a_b2,
               g2a_ln_g, g2a_ln_b, g2a_w1, g2a_b1, g2a_w2, g2a_b2,
               oth_ln_g, oth_ln_b, oth_w1, oth_b1, oth_w2, oth_b2,
               out_fc_W, out_fc_b,
               ffn_ln_g, ffn_ln_b, ffn_w1, ffn_b1, ffn_w3, ffn_b3,
               ffn_w2, ffn_b2)
    outs = [_dense_call(t * _BPT, agent_x, *gs[t], *weights)
            for t in range(3)]
    return jnp.concatenate(outs, axis=0)


# speedup vs baseline: 2.4059x; 2.4059x over previous
import functools

import jax
import jax.numpy as jnp
from jax import lax
from jax.experimental import pallas as pl
from jax.experimental.pallas import tpu as pltpu
from jax.experimental.pallas import tpu_sc as plsc

_H = 128
_NA = 90000
_NPT = 30000
_NW = 32
_CH = 128
_CPW = 8
_NP = _NW * _CPW * _CH
_B = 5000
_BPT = _NPT // _B


def _sc_gather3(lane_x, poly_x, agent_x, l2a_idx, g2a_idx, oth_idx, dep):
    info = plsc.get_sparse_core_info()
    nc = info.num_cores
    mesh = plsc.VectorSubcoreMesh(core_axis_name="c", subcore_axis_name="s")

    scratch = [
        pltpu.VMEM((_CPW, _CH), jnp.int32),
        pltpu.VMEM((_CH, _H), jnp.float32),
        pltpu.VMEM((_CH, _H), jnp.float32),
        pltpu.VMEM((_CH, _H), jnp.float32),
        pltpu.VMEM((_CH, _H), jnp.float32),
        pltpu.SemaphoreType.DMA,
        pltpu.SemaphoreType.DMA,
    ]

    def body(lane_hbm, poly_hbm, agent_hbm, li_hbm, gi_hbm, oi_hbm,
             lo_hbm, go_hbm, oo_hbm, idx_v, b0, b1, b2, b3, gsem, wsem):
        wid = lax.axis_index("s") * nc + lax.axis_index("c")
        base = wid * (_CPW * _CH)
        bufs = (b0, b1, b2, b3)
        for tab, ih, oh in ((lane_hbm, li_hbm, lo_hbm),
                            (poly_hbm, gi_hbm, go_hbm),
                            (agent_hbm, oi_hbm, oo_hbm)):
            pltpu.sync_copy(ih.at[wid], idx_v)

            def quad(i, carry, tab=tab, oh=oh):
                j = 4 * i
                gs = [pltpu.async_copy(tab.at[idx_v.at[j + b]], bufs[b], gsem)
                      for b in range(4)]
                ws = []
                for b in range(4):
                    gs[b].wait()
                    ws.append(pltpu.async_copy(
                        bufs[b], oh.at[pl.ds(base + (j + b) * _CH, _CH)],
                        wsem))
                for w in ws:
                    w.wait()
                return carry

            lax.fori_loop(0, _CPW // 4, quad, 0)

    out_type = [jax.ShapeDtypeStruct((_NP, _H), jnp.float32)] * 3
    if dep is None:
        k = functools.partial(pl.kernel, mesh=mesh, out_type=out_type,
                              scratch_types=scratch)(body)
        return k(lane_x, poly_x, agent_x, l2a_idx, g2a_idx, oth_idx)

    def body_dep(dep_hbm, *refs):
        body(*refs)

    k = functools.partial(pl.kernel, mesh=mesh, out_type=out_type,
                          scratch_types=scratch)(body_dep)
    return k(dep, lane_x, poly_x, agent_x, l2a_idx, g2a_idx, oth_idx)


def _ln(x, g, b):
    m = jnp.mean(x, axis=-1, keepdims=True)
    v = jnp.mean((x - m) * (x - m), axis=-1, keepdims=True)
    return (x - m) * lax.rsqrt(v + 1e-5) * g + b


def _dot(a, b):
    return lax.dot_general(a, b, (((1,), (0,)), ((), ())),
                           preferred_element_type=jnp.float32)


def _dense_body(cur_ref, lane_ref, poly_ref, oth_ref,
                wself_ref, bself_ref,
                l2a_g_ref, l2a_b_ref, l2a_w1_ref, l2a_b1_ref, l2a_w2_ref,
                l2a_b2_ref,
                g2a_g_ref, g2a_b_ref, g2a_w1_ref, g2a_b1_ref, g2a_w2_ref,
                g2a_b2_ref,
                oth_g_ref, oth_b_ref, oth_w1_ref, oth_b1_ref, oth_w2_ref,
                oth_b2_ref,
                ofc_w_ref, ofc_b_ref,
                ffn_g_ref, ffn_b_ref, ffn_w1_ref, ffn_b1_ref, ffn_w3_ref,
                ffn_b3_ref, ffn_w2_ref, ffn_b2_ref,
                out_ref):
    cur = cur_ref[...]
    lane = lane_ref[...]
    poly = poly_ref[...]
    oth = oth_ref[...]

    self_out = jnp.maximum(_dot(cur, wself_ref[0]) + bself_ref[0], 0.0)

    h = _ln(jnp.concatenate([lane, cur, lane], axis=-1),
            l2a_g_ref[...], l2a_b_ref[...])
    l2a = _dot(jnp.maximum(_dot(h, l2a_w1_ref[...]) + l2a_b1_ref[...], 0.0),
               l2a_w2_ref[...]) + l2a_b2_ref[...]

    h = _ln(jnp.concatenate([poly, cur], axis=-1),
            g2a_g_ref[...], g2a_b_ref[...])
    g2a = _dot(jnp.maximum(_dot(h, g2a_w1_ref[...]) + g2a_b1_ref[...], 0.0),
               g2a_w2_ref[...]) + g2a_b2_ref[...]

    h = _ln(jnp.concatenate([oth, cur, oth], axis=-1),
            oth_g_ref[0], oth_b_ref[0])
    oth_out = _dot(jnp.maximum(_dot(h, oth_w1_ref[0]) + oth_b1_ref[0], 0.0),
                   oth_w2_ref[0]) + oth_b2_ref[0]

    combined = jnp.concatenate([self_out, l2a, g2a, oth_out], axis=-1)
    x = _dot(combined, ofc_w_ref[0]) + ofc_b_ref[0]

    h = _ln(x, ffn_g_ref[0], ffn_b_ref[0])
    a = _dot(h, ffn_w1_ref[0]) + ffn_b1_ref[0]
    g = _dot(h, ffn_w3_ref[0]) + ffn_b3_ref[0]
    out_ref[...] = _dot(a * lax.logistic(a) * g, ffn_w2_ref[0]) \
        + ffn_b2_ref[0] + x


def _t3(v):
    return v.reshape(3, 1, -1)


def _dense_call(b0, cur, lane_g, poly_g, oth_g,
                W_self, b_self,
                l2a_g, l2a_b, l2a_w1, l2a_b1, l2a_w2, l2a_b2,
                g2a_g, g2a_b, g2a_w1, g2a_b1, g2a_w2, g2a_b2,
                oth_lg, oth_lb, oth_w1, oth_b1, oth_w2, oth_b2,
                ofc_w, ofc_b,
                ffn_g, ffn_b, ffn_w1, ffn_b1, ffn_w3, ffn_b3, ffn_w2, ffn_b2):
    rows = lambda i: (i, 0)
    rows_o = lambda i, b0=b0: (i + b0, 0)
    typ3 = lambda i, b0=b0: ((i + b0) // _BPT, 0, 0)
    full2 = lambda i: (0, 0)
    bs = pl.BlockSpec
    tvec = lambda n: bs((1, 1, n), typ3)

    in_specs = [
        bs((_B, _H), rows_o),
        bs((_B, _H), rows),
        bs((_B, _H), rows),
        bs((_B, _H), rows),
        bs((1, _H, _H), typ3),
        tvec(_H),
        bs((1, 3 * _H), full2),
        bs((1, 3 * _H), full2),
        bs((3 * _H, 4 * _H), full2),
        bs((1, 4 * _H), full2),
        bs((4 * _H, _H), full2),
        bs((1, _H), full2),
        bs((1, 2 * _H), full2),
        bs((1, 2 * _H), full2),
        bs((2 * _H, 4 * _H), full2),
        bs((1, 4 * _H), full2),
        bs((4 * _H, _H), full2),
        bs((1, _H), full2),
        tvec(3 * _H),
        tvec(3 * _H),
        bs((1, 3 * _H, 4 * _H), typ3),
        tvec(4 * _H),
        bs((1, 4 * _H, _H), typ3),
        tvec(_H),
        bs((1, 4 * _H, _H), typ3),
        tvec(_H),
        tvec(_H),
        tvec(_H),
        bs((1, _H, 4 * _H), typ3),
        tvec(4 * _H),
        bs((1, _H, 4 * _H), typ3),
        tvec(4 * _H),
        bs((1, 4 * _H, _H), typ3),
        tvec(_H),
    ]
    return pl.pallas_call(
        _dense_body,
        grid=(_BPT,),
        in_specs=in_specs,
        out_specs=bs((_B, _H), rows),
        out_shape=jax.ShapeDtypeStruct((_NPT, _H), jnp.float32),
        compiler_params=pltpu.CompilerParams(
            dimension_semantics=("arbitrary",)),
    )(cur, lane_g, poly_g, oth_g,
      W_self, _t3(b_self),
      l2a_g.reshape(1, -1), l2a_b.reshape(1, -1), l2a_w1,
      l2a_b1.reshape(1, -1), l2a_w2, l2a_b2.reshape(1, -1),
      g2a_g.reshape(1, -1), g2a_b.reshape(1, -1), g2a_w1,
      g2a_b1.reshape(1, -1), g2a_w2, g2a_b2.reshape(1, -1),
      _t3(oth_lg), _t3(oth_lb), oth_w1, _t3(oth_b1), oth_w2, _t3(oth_b2),
      ofc_w, _t3(ofc_b), _t3(ffn_g), _t3(ffn_b),
      ffn_w1, _t3(ffn_b1), ffn_w3, _t3(ffn_b3), ffn_w2, _t3(ffn_b2))


def _pad_idx(idx):
    idx = idx.astype(jnp.int32)
    pad = jnp.arange(_NP - _NPT, dtype=jnp.int32)
    return jnp.concatenate([idx, pad]).reshape(_NW, _CPW, _CH)


def kernel(agent_x, lane_x, polygon_x, W_self, b_self, l2a_ln_g, l2a_ln_b,
           l2a_w1, l2a_b1, l2a_w2, l2a_b2, g2a_ln_g, g2a_ln_b, g2a_w1,
           g2a_b1, g2a_w2, g2a_b2, oth_ln_g, oth_ln_b, oth_w1, oth_b1,
           oth_w2, oth_b2, out_fc_W, out_fc_b, ffn_ln_g, ffn_ln_b, ffn_w1,
           ffn_b1, ffn_w3, ffn_b3, ffn_w2, ffn_b2, l2a_src, g2a_src,
           other_src):
    gs = []
    dep = None
    for t in range(3):
        lo, hi = t * _NPT, (t + 1) * _NPT
        g = _sc_gather3(lane_x, polygon_x, agent_x,
                        _pad_idx(l2a_src[lo:hi]),
                        _pad_idx(g2a_src[lo:hi]),
                        _pad_idx(other_src[lo:hi]), dep)
        dep = g[0]
        gs.append(g)
    weights = (W_self, b_self,
               l2a_ln_g, l2a_ln_b, l2a_w1, l2a_b1, l2a_w2, l2a_b2,
               g2a_ln_g, g2a_ln_b, g2a_w1, g2a_b1, g2a_w2, g2a_b2,
               oth_ln_g, oth_ln_b, oth_w1, oth_b1, oth_w2, oth_b2,
               out_fc_W, out_fc_b,
               ffn_ln_g, ffn_ln_b, ffn_w1, ffn_b1, ffn_w3, ffn_b3,
               ffn_w2, ffn_b2)
    outs = [_dense_call(t * _BPT, agent_x, *gs[t], *weights)
            for t in range(3)]
    return jnp.concatenate(outs, axis=0)


# speedup vs baseline: 2.6824x; 1.1149x over previous
import functools

import jax
import jax.numpy as jnp
from jax import lax
from jax.experimental import pallas as pl
from jax.experimental.pallas import tpu as pltpu
from jax.experimental.pallas import tpu_sc as plsc

_H = 128
_NA = 90000
_NPT = 30000
_NW = 32
_CH = 128
_CPW = 8
_NP = _NW * _CPW * _CH
_B = 5000
_BPT = _NPT // _B


def _sc_gather3(lane_x, poly_x, agent_x, l2a_idx, g2a_idx, oth_idx, dep):
    info = plsc.get_sparse_core_info()
    nc = info.num_cores
    mesh = plsc.VectorSubcoreMesh(core_axis_name="c", subcore_axis_name="s")

    scratch = [
        pltpu.VMEM((_CPW, _CH), jnp.int32),
        pltpu.VMEM((_CH, _H), jnp.float32),
        pltpu.VMEM((_CH, _H), jnp.float32),
        pltpu.VMEM((_CH, _H), jnp.float32),
        pltpu.VMEM((_CH, _H), jnp.float32),
        pltpu.SemaphoreType.DMA,
        pltpu.SemaphoreType.DMA,
    ]

    def body(lane_hbm, poly_hbm, agent_hbm, li_hbm, gi_hbm, oi_hbm,
             lo_hbm, go_hbm, oo_hbm, idx_v, b0, b1, b2, b3, gsem, wsem):
        wid = lax.axis_index("s") * nc + lax.axis_index("c")
        base = wid * (_CPW * _CH)
        bufs = (b0, b1, b2, b3)
        for tab, ih, oh in ((lane_hbm, li_hbm, lo_hbm),
                            (poly_hbm, gi_hbm, go_hbm),
                            (agent_hbm, oi_hbm, oo_hbm)):
            pltpu.sync_copy(ih.at[wid], idx_v)

            def quad(i, carry, tab=tab, oh=oh):
                j = 4 * i
                gs = [pltpu.async_copy(tab.at[idx_v.at[j + b]], bufs[b], gsem)
                      for b in range(4)]
                ws = []
                for b in range(4):
                    gs[b].wait()
                    ws.append(pltpu.async_copy(
                        bufs[b], oh.at[pl.ds(base + (j + b) * _CH, _CH)],
                        wsem))
                for w in ws:
                    w.wait()
                return carry

            lax.fori_loop(0, _CPW // 4, quad, 0)

    out_type = [jax.ShapeDtypeStruct((_NP, _H), jnp.float32)] * 3
    if dep is None:
        k = functools.partial(pl.kernel, mesh=mesh, out_type=out_type,
                              scratch_types=scratch)(body)
        return k(lane_x, poly_x, agent_x, l2a_idx, g2a_idx, oth_idx)

    def body_dep(dep_hbm, *refs):
        body(*refs)

    k = functools.partial(pl.kernel, mesh=mesh, out_type=out_type,
                          scratch_types=scratch)(body_dep)
    return k(dep, lane_x, poly_x, agent_x, l2a_idx, g2a_idx, oth_idx)


def _ln(x, g, b):
    m = jnp.mean(x, axis=-1, keepdims=True)
    v = jnp.mean((x - m) * (x - m), axis=-1, keepdims=True)
    return (x - m) * lax.rsqrt(v + 1e-5) * g + b


def _dot(a, b):
    return lax.dot_general(a, b, (((1,), (0,)), ((), ())),
                           preferred_element_type=jnp.float32)


def _dense_body(cur_ref, lane_ref, poly_ref, oth_ref,
                wself_ref, bself_ref,
                l2a_g_ref, l2a_b_ref, l2a_w1_ref, l2a_b1_ref, l2a_w2_ref,
                l2a_b2_ref,
                g2a_g_ref, g2a_b_ref, g2a_w1_ref, g2a_b1_ref, g2a_w2_ref,
                g2a_b2_ref,
                oth_g_ref, oth_b_ref, oth_w1_ref, oth_b1_ref, oth_w2_ref,
                oth_b2_ref,
                ofc_w_ref, ofc_b_ref,
                ffn_g_ref, ffn_b_ref, ffn_w1_ref, ffn_b1_ref, ffn_w3_ref,
                ffn_b3_ref, ffn_w2_ref, ffn_b2_ref,
                out_ref):
    cur = cur_ref[...]
    lane = lane_ref[...]
    poly = poly_ref[...]
    oth = oth_ref[...]

    self_out = jnp.maximum(_dot(cur, wself_ref[0]) + bself_ref[0], 0.0)

    h = _ln(jnp.concatenate([lane, cur, lane], axis=-1),
            l2a_g_ref[...], l2a_b_ref[...])
    l2a = _dot(jnp.maximum(_dot(h, l2a_w1_ref[...]) + l2a_b1_ref[...], 0.0),
               l2a_w2_ref[...]) + l2a_b2_ref[...]

    h = _ln(jnp.concatenate([poly, cur], axis=-1),
            g2a_g_ref[...], g2a_b_ref[...])
    g2a = _dot(jnp.maximum(_dot(h, g2a_w1_ref[...]) + g2a_b1_ref[...], 0.0),
               g2a_w2_ref[...]) + g2a_b2_ref[...]

    h = _ln(jnp.concatenate([oth, cur, oth], axis=-1),
            oth_g_ref[0], oth_b_ref[0])
    oth_out = _dot(jnp.maximum(_dot(h, oth_w1_ref[0]) + oth_b1_ref[0], 0.0),
                   oth_w2_ref[0]) + oth_b2_ref[0]

    combined = jnp.concatenate([self_out, l2a, g2a, oth_out], axis=-1)
    x = _dot(combined, ofc_w_ref[0]) + ofc_b_ref[0]

    h = _ln(x, ffn_g_ref[0], ffn_b_ref[0])
    a = _dot(h, ffn_w1_ref[0]) + ffn_b1_ref[0]
    g = _dot(h, ffn_w3_ref[0]) + ffn_b3_ref[0]
    out_ref[...] = _dot(a * lax.logistic(a) * g, ffn_w2_ref[0]) \
        + ffn_b2_ref[0] + x


def _t3(v):
    return v.reshape(3, 1, -1)


def _dense_call(b0, cur, lane_g, poly_g, oth_g,
                W_self, b_self,
                l2a_g, l2a_b, l2a_w1, l2a_b1, l2a_w2, l2a_b2,
                g2a_g, g2a_b, g2a_w1, g2a_b1, g2a_w2, g2a_b2,
                oth_lg, oth_lb, oth_w1, oth_b1, oth_w2, oth_b2,
                ofc_w, ofc_b,
                ffn_g, ffn_b, ffn_w1, ffn_b1, ffn_w3, ffn_b3, ffn_w2, ffn_b2):
    rows = lambda i: (i, 0)
    rows_o = lambda i, b0=b0: (i + b0, 0)
    typ3 = lambda i, b0=b0: ((i + b0) // _BPT, 0, 0)
    full2 = lambda i: (0, 0)
    bs = pl.BlockSpec
    tvec = lambda n: bs((1, 1, n), typ3)

    in_specs = [
        bs((_B, _H), rows_o),
        bs((_B, _H), rows),
        bs((_B, _H), rows),
        bs((_B, _H), rows),
        bs((1, _H, _H), typ3),
        tvec(_H),
        bs((1, 3 * _H), full2),
        bs((1, 3 * _H), full2),
        bs((3 * _H, 4 * _H), full2),
        bs((1, 4 * _H), full2),
        bs((4 * _H, _H), full2),
        bs((1, _H), full2),
        bs((1, 2 * _H), full2),
        bs((1, 2 * _H), full2),
        bs((2 * _H, 4 * _H), full2),
        bs((1, 4 * _H), full2),
        bs((4 * _H, _H), full2),
        bs((1, _H), full2),
        tvec(3 * _H),
        tvec(3 * _H),
        bs((1, 3 * _H, 4 * _H), typ3),
        tvec(4 * _H),
        bs((1, 4 * _H, _H), typ3),
        tvec(_H),
        bs((1, 4 * _H, _H), typ3),
        tvec(_H),
        tvec(_H),
        tvec(_H),
        bs((1, _H, 4 * _H), typ3),
        tvec(4 * _H),
        bs((1, _H, 4 * _H), typ3),
        tvec(4 * _H),
        bs((1, 4 * _H, _H), typ3),
        tvec(_H),
    ]
    return pl.pallas_call(
        _dense_body,
        grid=(_BPT,),
        in_specs=in_specs,
        out_specs=bs((_B, _H), rows),
        out_shape=jax.ShapeDtypeStruct((_NPT, _H), jnp.float32),
        compiler_params=pltpu.CompilerParams(
            dimension_semantics=("arbitrary",)),
    )(cur, lane_g, poly_g, oth_g,
      W_self, _t3(b_self),
      l2a_g.reshape(1, -1), l2a_b.reshape(1, -1), l2a_w1,
      l2a_b1.reshape(1, -1), l2a_w2, l2a_b2.reshape(1, -1),
      g2a_g.reshape(1, -1), g2a_b.reshape(1, -1), g2a_w1,
      g2a_b1.reshape(1, -1), g2a_w2, g2a_b2.reshape(1, -1),
      _t3(oth_lg), _t3(oth_lb), oth_w1, _t3(oth_b1), oth_w2, _t3(oth_b2),
      ofc_w, _t3(ofc_b), _t3(ffn_g), _t3(ffn_b),
      ffn_w1, _t3(ffn_b1), ffn_w3, _t3(ffn_b3), ffn_w2, _t3(ffn_b2))


def _pad_idx(idx):
    idx = idx.astype(jnp.int32)
    pad = jnp.arange(_NP - _NPT, dtype=jnp.int32)
    return jnp.concatenate([idx, pad]).reshape(_NW, _CPW, _CH)


def kernel(agent_x, lane_x, polygon_x, W_self, b_self, l2a_ln_g, l2a_ln_b,
           l2a_w1, l2a_b1, l2a_w2, l2a_b2, g2a_ln_g, g2a_ln_b, g2a_w1,
           g2a_b1, g2a_w2, g2a_b2, oth_ln_g, oth_ln_b, oth_w1, oth_b1,
           oth_w2, oth_b2, out_fc_W, out_fc_b, ffn_ln_g, ffn_ln_b, ffn_w1,
           ffn_b1, ffn_w3, ffn_b3, ffn_w2, ffn_b2, l2a_src, g2a_src,
           other_src):
    gs = []
    dep = None
    for t in range(3):
        lo, hi = t * _NPT, (t + 1) * _NPT
        g = _sc_gather3(lane_x, polygon_x, agent_x,
                        _pad_idx(l2a_src[lo:hi]),
                        _pad_idx(g2a_src[lo:hi]),
                        _pad_idx(other_src[lo:hi]), None)
        gs.append(g)
    weights = (W_self, b_self,
               l2a_ln_g, l2a_ln_b, l2a_w1, l2a_b1, l2a_w2, l2a_b2,
               g2a_ln_g, g2a_ln_b, g2a_w1, g2a_b1, g2a_w2, g2a_b2,
               oth_ln_g, oth_ln_b, oth_w1, oth_b1, oth_w2, oth_b2,
               out_fc_W, out_fc_b,
               ffn_ln_g, ffn_ln_b, ffn_w1, ffn_b1, ffn_w3, ffn_b3,
               ffn_w2, ffn_b2)
    outs = [_dense_call(t * _BPT, agent_x, *gs[t], *weights)
            for t in range(3)]
    return jnp.concatenate(outs, axis=0)


# speedup vs baseline: 2.8955x; 1.0795x over previous
import functools

import jax
import jax.numpy as jnp
from jax import lax
from jax.experimental import pallas as pl
from jax.experimental.pallas import tpu as pltpu
from jax.experimental.pallas import tpu_sc as plsc

_H = 128
_NA = 90000
_NPT = 30000
_NW = 32
_CH = 128
_CPW = 8
_NP = _NW * _CPW * _CH
_B = 5000
_BPT = _NPT // _B


def _sc_gather3(lane_x, poly_x, agent_x, l2a_idx, g2a_idx, oth_idx, dep):
    info = plsc.get_sparse_core_info()
    nc = info.num_cores
    mesh = plsc.VectorSubcoreMesh(core_axis_name="c", subcore_axis_name="s")

    scratch = [
        pltpu.VMEM((_CPW, _CH), jnp.int32),
        pltpu.VMEM((_CH, _H), jnp.float32),
        pltpu.VMEM((_CH, _H), jnp.float32),
        pltpu.VMEM((_CH, _H), jnp.float32),
        pltpu.VMEM((_CH, _H), jnp.float32),
        pltpu.SemaphoreType.DMA,
        pltpu.SemaphoreType.DMA,
    ]

    def body(lane_hbm, poly_hbm, agent_hbm, li_hbm, gi_hbm, oi_hbm,
             lo_hbm, go_hbm, oo_hbm, idx_v, b0, b1, b2, b3, gsem, wsem):
        wid = lax.axis_index("s") * nc + lax.axis_index("c")
        base = wid * (_CPW * _CH)
        bufs = (b0, b1, b2, b3)
        for tab, ih, oh in ((lane_hbm, li_hbm, lo_hbm),
                            (poly_hbm, gi_hbm, go_hbm),
                            (agent_hbm, oi_hbm, oo_hbm)):
            pltpu.sync_copy(ih.at[wid], idx_v)

            def quad(i, carry, tab=tab, oh=oh):
                j = 4 * i
                gs = [pltpu.async_copy(tab.at[idx_v.at[j + b]], bufs[b], gsem)
                      for b in range(4)]
                ws = []
                for b in range(4):
                    gs[b].wait()
                    ws.append(pltpu.async_copy(
                        bufs[b], oh.at[pl.ds(base + (j + b) * _CH, _CH)],
                        wsem))
                for w in ws:
                    w.wait()
                return carry

            lax.fori_loop(0, _CPW // 4, quad, 0)

    out_type = [jax.ShapeDtypeStruct((_NP, _H), jnp.float32)] * 3
    if dep is None:
        k = functools.partial(pl.kernel, mesh=mesh, out_type=out_type,
                              scratch_types=scratch)(body)
        return k(lane_x, poly_x, agent_x, l2a_idx, g2a_idx, oth_idx)

    def body_dep(dep_hbm, *refs):
        body(*refs)

    k = functools.partial(pl.kernel, mesh=mesh, out_type=out_type,
                          scratch_types=scratch)(body_dep)
    return k(dep, lane_x, poly_x, agent_x, l2a_idx, g2a_idx, oth_idx)


def _ln(x, g, b):
    m = jnp.mean(x, axis=-1, keepdims=True)
    v = jnp.mean((x - m) * (x - m), axis=-1, keepdims=True)
    return (x - m) * lax.rsqrt(v + 1e-5) * g + b


def _dot(a, b):
    return lax.dot_general(a, b, (((1,), (0,)), ((), ())),
                           preferred_element_type=jnp.float32)


def _dense_body(cur_ref, lane_ref, poly_ref, oth_ref,
                wself_ref, bself_ref,
                l2a_g_ref, l2a_b_ref, l2a_w1_ref, l2a_b1_ref, l2a_w2_ref,
                l2a_b2_ref,
                g2a_g_ref, g2a_b_ref, g2a_w1_ref, g2a_b1_ref, g2a_w2_ref,
                g2a_b2_ref,
                oth_g_ref, oth_b_ref, oth_w1_ref, oth_b1_ref, oth_w2_ref,
                oth_b2_ref,
                ofc_w_ref, ofc_b_ref,
                ffn_g_ref, ffn_b_ref, ffn_w1_ref, ffn_b1_ref, ffn_w3_ref,
                ffn_b3_ref, ffn_w2_ref, ffn_b2_ref,
                out_ref):
    cur = cur_ref[...]
    lane = lane_ref[...]
    poly = poly_ref[...]
    oth = oth_ref[...]

    self_out = jnp.maximum(_dot(cur, wself_ref[0]) + bself_ref[0], 0.0)

    h = _ln(jnp.concatenate([lane, cur, lane], axis=-1),
            l2a_g_ref[...], l2a_b_ref[...])
    l2a = _dot(jnp.maximum(_dot(h, l2a_w1_ref[...]) + l2a_b1_ref[...], 0.0),
               l2a_w2_ref[...]) + l2a_b2_ref[...]

    h = _ln(jnp.concatenate([poly, cur], axis=-1),
            g2a_g_ref[...], g2a_b_ref[...])
    g2a = _dot(jnp.maximum(_dot(h, g2a_w1_ref[...]) + g2a_b1_ref[...], 0.0),
               g2a_w2_ref[...]) + g2a_b2_ref[...]

    h = _ln(jnp.concatenate([oth, cur, oth], axis=-1),
            oth_g_ref[0], oth_b_ref[0])
    oth_out = _dot(jnp.maximum(_dot(h, oth_w1_ref[0]) + oth_b1_ref[0], 0.0),
                   oth_w2_ref[0]) + oth_b2_ref[0]

    combined = jnp.concatenate([self_out, l2a, g2a, oth_out], axis=-1)
    x = _dot(combined, ofc_w_ref[0]) + ofc_b_ref[0]

    h = _ln(x, ffn_g_ref[0], ffn_b_ref[0])
    a = _dot(h, ffn_w1_ref[0]) + ffn_b1_ref[0]
    g = _dot(h, ffn_w3_ref[0]) + ffn_b3_ref[0]
    out_ref[...] = _dot(a * lax.logistic(a) * g, ffn_w2_ref[0]) \
        + ffn_b2_ref[0] + x


def _t3(v):
    return v.reshape(3, 1, -1)


def _dense_body_alias(prev_ref, *refs):
    _dense_body(*refs)


def _dense_call(b0, prev, cur, lane_g, poly_g, oth_g,
                W_self, b_self,
                l2a_g, l2a_b, l2a_w1, l2a_b1, l2a_w2, l2a_b2,
                g2a_g, g2a_b, g2a_w1, g2a_b1, g2a_w2, g2a_b2,
                oth_lg, oth_lb, oth_w1, oth_b1, oth_w2, oth_b2,
                ofc_w, ofc_b,
                ffn_g, ffn_b, ffn_w1, ffn_b1, ffn_w3, ffn_b3, ffn_w2, ffn_b2):
    rows = lambda i: (i, 0)
    rows_o = lambda i, b0=b0: (i + b0, 0)
    typ3 = lambda i, b0=b0: ((i + b0) // _BPT, 0, 0)
    full2 = lambda i: (0, 0)
    bs = pl.BlockSpec
    tvec = lambda n: bs((1, 1, n), typ3)

    in_specs = [
        bs((_B, _H), rows_o),
        bs((_B, _H), rows),
        bs((_B, _H), rows),
        bs((_B, _H), rows),
        bs((1, _H, _H), typ3),
        tvec(_H),
        bs((1, 3 * _H), full2),
        bs((1, 3 * _H), full2),
        bs((3 * _H, 4 * _H), full2),
        bs((1, 4 * _H), full2),
        bs((4 * _H, _H), full2),
        bs((1, _H), full2),
        bs((1, 2 * _H), full2),
        bs((1, 2 * _H), full2),
        bs((2 * _H, 4 * _H), full2),
        bs((1, 4 * _H), full2),
        bs((4 * _H, _H), full2),
        bs((1, _H), full2),
        tvec(3 * _H),
        tvec(3 * _H),
        bs((1, 3 * _H, 4 * _H), typ3),
        tvec(4 * _H),
        bs((1, 4 * _H, _H), typ3),
        tvec(_H),
        bs((1, 4 * _H, _H), typ3),
        tvec(_H),
        tvec(_H),
        tvec(_H),
        bs((1, _H, 4 * _H), typ3),
        tvec(4 * _H),
        bs((1, _H, 4 * _H), typ3),
        tvec(4 * _H),
        bs((1, 4 * _H, _H), typ3),
        tvec(_H),
    ]
    body = _dense_body
    alias = {}
    args_pre = ()
    if prev is not None:
        body = _dense_body_alias
        alias = {0: 0}
        in_specs = [bs(memory_space=pl.ANY)] + in_specs
        args_pre = (prev,)
    return pl.pallas_call(
        body,
        grid=(_BPT,),
        in_specs=in_specs,
        out_specs=bs((_B, _H), rows_o),
        out_shape=jax.ShapeDtypeStruct((_NA, _H), jnp.float32),
        input_output_aliases=alias,
        compiler_params=pltpu.CompilerParams(
            dimension_semantics=("arbitrary",)),
    )(*args_pre, cur, lane_g, poly_g, oth_g,
      W_self, _t3(b_self),
      l2a_g.reshape(1, -1), l2a_b.reshape(1, -1), l2a_w1,
      l2a_b1.reshape(1, -1), l2a_w2, l2a_b2.reshape(1, -1),
      g2a_g.reshape(1, -1), g2a_b.reshape(1, -1), g2a_w1,
      g2a_b1.reshape(1, -1), g2a_w2, g2a_b2.reshape(1, -1),
      _t3(oth_lg), _t3(oth_lb), oth_w1, _t3(oth_b1), oth_w2, _t3(oth_b2),
      ofc_w, _t3(ofc_b), _t3(ffn_g), _t3(ffn_b),
      ffn_w1, _t3(ffn_b1), ffn_w3, _t3(ffn_b3), ffn_w2, _t3(ffn_b2))


def _pad_idx(idx):
    idx = idx.astype(jnp.int32)
    pad = jnp.arange(_NP - _NPT, dtype=jnp.int32)
    return jnp.concatenate([idx, pad]).reshape(_NW, _CPW, _CH)


def kernel(agent_x, lane_x, polygon_x, W_self, b_self, l2a_ln_g, l2a_ln_b,
           l2a_w1, l2a_b1, l2a_w2, l2a_b2, g2a_ln_g, g2a_ln_b, g2a_w1,
           g2a_b1, g2a_w2, g2a_b2, oth_ln_g, oth_ln_b, oth_w1, oth_b1,
           oth_w2, oth_b2, out_fc_W, out_fc_b, ffn_ln_g, ffn_ln_b, ffn_w1,
           ffn_b1, ffn_w3, ffn_b3, ffn_w2, ffn_b2, l2a_src, g2a_src,
           other_src):
    gs = []
    dep = None
    for t in range(3):
        lo, hi = t * _NPT, (t + 1) * _NPT
        g = _sc_gather3(lane_x, polygon_x, agent_x,
                        _pad_idx(l2a_src[lo:hi]),
                        _pad_idx(g2a_src[lo:hi]),
                        _pad_idx(other_src[lo:hi]), None)
        gs.append(g)
    weights = (W_self, b_self,
               l2a_ln_g, l2a_ln_b, l2a_w1, l2a_b1, l2a_w2, l2a_b2,
               g2a_ln_g, g2a_ln_b, g2a_w1, g2a_b1, g2a_w2, g2a_b2,
               oth_ln_g, oth_ln_b, oth_w1, oth_b1, oth_w2, oth_b2,
               out_fc_W, out_fc_b,
               ffn_ln_g, ffn_ln_b, ffn_w1, ffn_b1, ffn_w3, ffn_b3,
               ffn_w2, ffn_b2)
    out = None
    for t in range(3):
        out = _dense_call(t * _BPT, out, agent_x, *gs[t], *weights)
    return out


# speedup vs baseline: 3.0776x; 1.0629x over previous
import functools

import jax
import jax.numpy as jnp
from jax import lax
from jax.experimental import pallas as pl
from jax.experimental.pallas import tpu as pltpu
from jax.experimental.pallas import tpu_sc as plsc

_H = 128
_NA = 90000
_NPT = 30000
_NW = 32
_CH = 128
_CPW = 8
_NP = _NW * _CPW * _CH
_B = 3000
_BPT = _NPT // _B


def _sc_gather3(lane_x, poly_x, agent_x, l2a_idx, g2a_idx, oth_idx, dep):
    info = plsc.get_sparse_core_info()
    nc = info.num_cores
    mesh = plsc.VectorSubcoreMesh(core_axis_name="c", subcore_axis_name="s")

    scratch = [
        pltpu.VMEM((_CPW, _CH), jnp.int32),
        pltpu.VMEM((_CH, _H), jnp.float32),
        pltpu.VMEM((_CH, _H), jnp.float32),
        pltpu.VMEM((_CH, _H), jnp.float32),
        pltpu.VMEM((_CH, _H), jnp.float32),
        pltpu.SemaphoreType.DMA,
        pltpu.SemaphoreType.DMA,
    ]

    def body(lane_hbm, poly_hbm, agent_hbm, li_hbm, gi_hbm, oi_hbm,
             lo_hbm, go_hbm, oo_hbm, idx_v, b0, b1, b2, b3, gsem, wsem):
        wid = lax.axis_index("s") * nc + lax.axis_index("c")
        base = wid * (_CPW * _CH)
        bufs = (b0, b1, b2, b3)
        for tab, ih, oh in ((lane_hbm, li_hbm, lo_hbm),
                            (poly_hbm, gi_hbm, go_hbm),
                            (agent_hbm, oi_hbm, oo_hbm)):
            pltpu.sync_copy(ih.at[wid], idx_v)

            def quad(i, carry, tab=tab, oh=oh):
                j = 4 * i
                gs = [pltpu.async_copy(tab.at[idx_v.at[j + b]], bufs[b], gsem)
                      for b in range(4)]
                ws = []
                for b in range(4):
                    gs[b].wait()
                    ws.append(pltpu.async_copy(
                        bufs[b], oh.at[pl.ds(base + (j + b) * _CH, _CH)],
                        wsem))
                for w in ws:
                    w.wait()
                return carry

            lax.fori_loop(0, _CPW // 4, quad, 0)

    out_type = [jax.ShapeDtypeStruct((_NP, _H), jnp.float32)] * 3
    if dep is None:
        k = functools.partial(pl.kernel, mesh=mesh, out_type=out_type,
                              scratch_types=scratch)(body)
        return k(lane_x, poly_x, agent_x, l2a_idx, g2a_idx, oth_idx)

    def body_dep(dep_hbm, *refs):
        body(*refs)

    k = functools.partial(pl.kernel, mesh=mesh, out_type=out_type,
                          scratch_types=scratch)(body_dep)
    return k(dep, lane_x, poly_x, agent_x, l2a_idx, g2a_idx, oth_idx)


def _ln(x, g, b):
    m = jnp.mean(x, axis=-1, keepdims=True)
    v = jnp.mean((x - m) * (x - m), axis=-1, keepdims=True)
    return (x - m) * lax.rsqrt(v + 1e-5) * g + b


def _dot(a, b):
    return lax.dot_general(a, b, (((1,), (0,)), ((), ())),
                           preferred_element_type=jnp.float32)


def _dense_body(cur_ref, lane_ref, poly_ref, oth_ref,
                wself_ref, bself_ref,
                l2a_g_ref, l2a_b_ref, l2a_w1_ref, l2a_b1_ref, l2a_w2_ref,
                l2a_b2_ref,
                g2a_g_ref, g2a_b_ref, g2a_w1_ref, g2a_b1_ref, g2a_w2_ref,
                g2a_b2_ref,
                oth_g_ref, oth_b_ref, oth_w1_ref, oth_b1_ref, oth_w2_ref,
                oth_b2_ref,
                ofc_w_ref, ofc_b_ref,
                ffn_g_ref, ffn_b_ref, ffn_w1_ref, ffn_b1_ref, ffn_w3_ref,
                ffn_b3_ref, ffn_w2_ref, ffn_b2_ref,
                out_ref):
    cur = cur_ref[...]
    lane = lane_ref[...]
    poly = poly_ref[...]
    oth = oth_ref[...]

    self_out = jnp.maximum(_dot(cur, wself_ref[0]) + bself_ref[0], 0.0)

    h = _ln(jnp.concatenate([lane, cur, lane], axis=-1),
            l2a_g_ref[...], l2a_b_ref[...])
    l2a = _dot(jnp.maximum(_dot(h, l2a_w1_ref[...]) + l2a_b1_ref[...], 0.0),
               l2a_w2_ref[...]) + l2a_b2_ref[...]

    h = _ln(jnp.concatenate([poly, cur], axis=-1),
            g2a_g_ref[...], g2a_b_ref[...])
    g2a = _dot(jnp.maximum(_dot(h, g2a_w1_ref[...]) + g2a_b1_ref[...], 0.0),
               g2a_w2_ref[...]) + g2a_b2_ref[...]

    h = _ln(jnp.concatenate([oth, cur, oth], axis=-1),
            oth_g_ref[0], oth_b_ref[0])
    oth_out = _dot(jnp.maximum(_dot(h, oth_w1_ref[0]) + oth_b1_ref[0], 0.0),
                   oth_w2_ref[0]) + oth_b2_ref[0]

    combined = jnp.concatenate([self_out, l2a, g2a, oth_out], axis=-1)
    x = _dot(combined, ofc_w_ref[0]) + ofc_b_ref[0]

    h = _ln(x, ffn_g_ref[0], ffn_b_ref[0])
    a = _dot(h, ffn_w1_ref[0]) + ffn_b1_ref[0]
    g = _dot(h, ffn_w3_ref[0]) + ffn_b3_ref[0]
    out_ref[...] = _dot(a * lax.logistic(a) * g, ffn_w2_ref[0]) \
        + ffn_b2_ref[0] + x


def _t3(v):
    return v.reshape(3, 1, -1)


def _dense_body_alias(prev_ref, *refs):
    _dense_body(*refs)


def _dense_call(b0, prev, cur, lane_g, poly_g, oth_g,
                W_self, b_self,
                l2a_g, l2a_b, l2a_w1, l2a_b1, l2a_w2, l2a_b2,
                g2a_g, g2a_b, g2a_w1, g2a_b1, g2a_w2, g2a_b2,
                oth_lg, oth_lb, oth_w1, oth_b1, oth_w2, oth_b2,
                ofc_w, ofc_b,
                ffn_g, ffn_b, ffn_w1, ffn_b1, ffn_w3, ffn_b3, ffn_w2, ffn_b2):
    rows = lambda i: (i, 0)
    rows_o = lambda i, b0=b0: (i + b0, 0)
    typ3 = lambda i, b0=b0: ((i + b0) // _BPT, 0, 0)
    full2 = lambda i: (0, 0)
    bs = pl.BlockSpec
    tvec = lambda n: bs((1, 1, n), typ3)

    in_specs = [
        bs((_B, _H), rows_o),
        bs((_B, _H), rows),
        bs((_B, _H), rows),
        bs((_B, _H), rows),
        bs((1, _H, _H), typ3),
        tvec(_H),
        bs((1, 3 * _H), full2),
        bs((1, 3 * _H), full2),
        bs((3 * _H, 4 * _H), full2),
        bs((1, 4 * _H), full2),
        bs((4 * _H, _H), full2),
        bs((1, _H), full2),
        bs((1, 2 * _H), full2),
        bs((1, 2 * _H), full2),
        bs((2 * _H, 4 * _H), full2),
        bs((1, 4 * _H), full2),
        bs((4 * _H, _H), full2),
        bs((1, _H), full2),
        tvec(3 * _H),
        tvec(3 * _H),
        bs((1, 3 * _H, 4 * _H), typ3),
        tvec(4 * _H),
        bs((1, 4 * _H, _H), typ3),
        tvec(_H),
        bs((1, 4 * _H, _H), typ3),
        tvec(_H),
        tvec(_H),
        tvec(_H),
        bs((1, _H, 4 * _H), typ3),
        tvec(4 * _H),
        bs((1, _H, 4 * _H), typ3),
        tvec(4 * _H),
        bs((1, 4 * _H, _H), typ3),
        tvec(_H),
    ]
    body = _dense_body
    alias = {}
    args_pre = ()
    if prev is not None:
        body = _dense_body_alias
        alias = {0: 0}
        in_specs = [bs(memory_space=pl.ANY)] + in_specs
        args_pre = (prev,)
    return pl.pallas_call(
        body,
        grid=(_BPT,),
        in_specs=in_specs,
        out_specs=bs((_B, _H), rows_o),
        out_shape=jax.ShapeDtypeStruct((_NA, _H), jnp.float32),
        input_output_aliases=alias,
        compiler_params=pltpu.CompilerParams(
            dimension_semantics=("arbitrary",)),
    )(*args_pre, cur, lane_g, poly_g, oth_g,
      W_self, _t3(b_self),
      l2a_g.reshape(1, -1), l2a_b.reshape(1, -1), l2a_w1,
      l2a_b1.reshape(1, -1), l2a_w2, l2a_b2.reshape(1, -1),
      g2a_g.reshape(1, -1), g2a_b.reshape(1, -1), g2a_w1,
      g2a_b1.reshape(1, -1), g2a_w2, g2a_b2.reshape(1, -1),
      _t3(oth_lg), _t3(oth_lb), oth_w1, _t3(oth_b1), oth_w2, _t3(oth_b2),
      ofc_w, _t3(ofc_b), _t3(ffn_g), _t3(ffn_b),
      ffn_w1, _t3(ffn_b1), ffn_w3, _t3(ffn_b3), ffn_w2, _t3(ffn_b2))


def _pad_idx(idx):
    idx = idx.astype(jnp.int32)
    pad = jnp.arange(_NP - _NPT, dtype=jnp.int32)
    return jnp.concatenate([idx, pad]).reshape(_NW, _CPW, _CH)


def kernel(agent_x, lane_x, polygon_x, W_self, b_self, l2a_ln_g, l2a_ln_b,
           l2a_w1, l2a_b1, l2a_w2, l2a_b2, g2a_ln_g, g2a_ln_b, g2a_w1,
           g2a_b1, g2a_w2, g2a_b2, oth_ln_g, oth_ln_b, oth_w1, oth_b1,
           oth_w2, oth_b2, out_fc_W, out_fc_b, ffn_ln_g, ffn_ln_b, ffn_w1,
           ffn_b1, ffn_w3, ffn_b3, ffn_w2, ffn_b2, l2a_src, g2a_src,
           other_src):
    gs = []
    dep = None
    for t in range(3):
        lo, hi = t * _NPT, (t + 1) * _NPT
        g = _sc_gather3(lane_x, polygon_x, agent_x,
                        _pad_idx(l2a_src[lo:hi]),
                        _pad_idx(g2a_src[lo:hi]),
                        _pad_idx(other_src[lo:hi]), None)
        gs.append(g)
    weights = (W_self, b_self,
               l2a_ln_g, l2a_ln_b, l2a_w1, l2a_b1, l2a_w2, l2a_b2,
               g2a_ln_g, g2a_ln_b, g2a_w1, g2a_b1, g2a_w2, g2a_b2,
               oth_ln_g, oth_ln_b, oth_w1, oth_b1, oth_w2, oth_b2,
               out_fc_W, out_fc_b,
               ffn_ln_g, ffn_ln_b, ffn_w1, ffn_b1, ffn_w3, ffn_b3,
               ffn_w2, ffn_b2)
    out = None
    for t in range(3):
        out = _dense_call(t * _BPT, out, agent_x, *gs[t], *weights)
    return out
